# baseline (speedup 1.0000x reference)
import functools

import jax
import jax.numpy as jnp
from jax import lax
from jax.experimental import pallas as pl
from jax.experimental.pallas import tpu as pltpu
from jax.experimental.pallas import tpu_sc as plsc

_B = 16384
_EMB = 128
_NC = 2
_NS = 16
_NW = _NC * _NS
_BPW = _B // _NW
_C = 128
_NCHUNK = _BPW // _C
_L = 16


def _build():
    mesh = plsc.VectorSubcoreMesh(core_axis_name="c", subcore_axis_name="s")

    @functools.partial(
        pl.kernel,
        mesh=mesh,
        compiler_params=pltpu.CompilerParams(needs_layout_passes=False),
        out_type=jax.ShapeDtypeStruct((_B,), jnp.float32),
        scratch_types=[
            pltpu.VMEM((_BPW,), jnp.int32),
            pltpu.VMEM((_BPW,), jnp.int32),
            pltpu.VMEM((_BPW,), jnp.int32),
            pltpu.VMEM((_C, _EMB), jnp.float32),
            pltpu.VMEM((_C, _EMB), jnp.float32),
            pltpu.VMEM((_C, _EMB), jnp.float32),
            pltpu.VMEM((_C, _EMB), jnp.float32),
            pltpu.VMEM((_C, _EMB), jnp.float32),
            pltpu.VMEM((_C, _EMB), jnp.float32),
            pltpu.VMEM((_L * _L,), jnp.float32),
            pltpu.VMEM((_BPW,), jnp.float32),
            pltpu.SemaphoreType.DMA,
            pltpu.SemaphoreType.DMA,
        ],
    )
    def transe(h_hbm, r_hbm, t_hbm, ent_hbm, rel_hbm, out_hbm,
               hidx, ridx, tidx, hrow0, rrow0, trow0, hrow1, rrow1, trow1,
               accbuf, scores, sem0, sem1):
        wid = lax.axis_index("s") * _NC + lax.axis_index("c")
        base = wid * _BPW
        pltpu.sync_copy(h_hbm.at[pl.ds(base, _BPW)], hidx)
        pltpu.sync_copy(r_hbm.at[pl.ds(base, _BPW)], ridx)
        pltpu.sync_copy(t_hbm.at[pl.ds(base, _BPW)], tidx)

        bufs = ((hrow0, rrow0, trow0, sem0), (hrow1, rrow1, trow1, sem1))
        lanes = lax.iota(jnp.int32, _L)

        def issue(ci):
            hrow, rrow, trow, sem = bufs[ci % 2]
            off = ci * _C
            dh = pltpu.async_copy(ent_hbm.at[hidx.at[pl.ds(off, _C)]], hrow, sem)
            dt = pltpu.async_copy(ent_hbm.at[tidx.at[pl.ds(off, _C)]], trow, sem)
            dr = pltpu.async_copy(rel_hbm.at[ridx.at[pl.ds(off, _C)]], rrow, sem)
            return (dh, dt, dr)

        def compute(ci):
            hrow, rrow, trow, _ = bufs[ci % 2]
            off = ci * _C

            def group(g, carry):
                rbase = g * _L
                for i in range(_L):
                    acc = jnp.zeros((_L,), jnp.float32)
                    for j in range(_EMB // _L):
                        hv = hrow[rbase + i, pl.ds(j * _L, _L)]
                        rv = rrow[rbase + i, pl.ds(j * _L, _L)]
                        tv = trow[rbase + i, pl.ds(j * _L, _L)]
                        d = (hv + rv) - tv
                        acc = acc + d * d
                    accbuf[pl.ds(i * _L, _L)] = acc
                sv = jnp.zeros((_L,), jnp.float32)
                for k in range(_L):
                    sv = sv + plsc.load_gather(accbuf, [lanes * _L + k])
                scores[pl.ds(off + g * _L, _L)] = sv
                return carry

            lax.fori_loop(0, _C // _L, group, 0)

        descs = issue(0)
        for ci in range(_NCHUNK):
            if ci + 1 < _NCHUNK:
                nxt = issue(ci + 1)
            for d in descs:
                d.wait()
            compute(ci)
            if ci + 1 < _NCHUNK:
                descs = nxt

        pltpu.sync_copy(scores, out_hbm.at[pl.ds(base, _BPW)])

    return transe


_TRANSE = _build()


def kernel(h, r, t, ent_emb, rel_emb):
    return _TRANSE(h.astype(jnp.int32), r.astype(jnp.int32),
                   t.astype(jnp.int32), ent_emb, rel_emb)
